# Initial kernel scaffold; baseline (speedup 1.0000x reference)
import jax
import jax.numpy as jnp
from jax.experimental import pallas as pl


def kernel(x, edge_attr, edge_index, batch_ids, params):
    raise NotImplementedError("write your pallas kernel here")


# speedup vs baseline: 4.6632x; 4.6632x over previous
import functools

import jax
import jax.numpy as jnp
from jax import lax
from jax.experimental import pallas as pl
from jax.experimental.pallas import tpu as pltpu
from jax.experimental.pallas import tpu_sc as plsc

D = 128
G = 64
HALF = 64
W = 128
TILES = 16
NB = 512


@functools.lru_cache(maxsize=None)
def _make_edge_kernel(NP, EP):
    per_tile = EP // TILES
    chunks = per_tile // W
    rows_pt = NP // TILES
    mesh = plsc.VectorSubcoreMesh(core_axis_name="c", subcore_axis_name="s")
    out_t = (jax.ShapeDtypeStruct((NP, 2, HALF), jnp.float32),
             jax.ShapeDtypeStruct((NP, 2, HALF), jnp.float32))
    scratch = [
        pltpu.VMEM((W,), jnp.int32),
        pltpu.VMEM((W,), jnp.int32),
        pltpu.VMEM((W,), jnp.int32),
        pltpu.VMEM((W,), jnp.int32),
        pltpu.VMEM((W, HALF), jnp.float32),
        pltpu.VMEM((W, HALF), jnp.float32),
        pltpu.VMEM((W, HALF), jnp.float32),
        pltpu.VMEM((W, HALF), jnp.float32),
        pltpu.VMEM((W, HALF), jnp.float32),
        pltpu.VMEM((64, HALF), jnp.float32),
        pltpu.VMEM_SHARED((NP, HALF), jnp.float32),
        pltpu.VMEM_SHARED((NP, HALF), jnp.float32),
        pltpu.SemaphoreType.DMA,
    ]

    @functools.partial(
        pl.kernel, out_type=out_t, mesh=mesh, scratch_types=scratch,
        compiler_params=pltpu.CompilerParams(use_tc_tiling_on_sc=False))
    def edge_kernel(srcp, dstp, ex2, dx2, bx2, num_out, den_out,
                    isrc, idst, i2s, i2d, rE, rD, rB, cn, cd, zb,
                    accn, accd, sem):
        c = lax.axis_index("c")
        s = lax.axis_index("s")

        zv = jnp.zeros((16,), jnp.float32)

        def zrow(i, _):
            for f in range(HALF // 16):
                zb[i, pl.ds(f * 16, 16)] = zv
            return 0

        lax.fori_loop(0, 64, zrow, 0)
        r0 = s * rows_pt
        for k in range(rows_pt // 64):
            pltpu.sync_copy(zb, accn.at[pl.ds(r0 + k * 64, 64)])
            pltpu.sync_copy(zb, accd.at[pl.ds(r0 + k * 64, 64)])
        plsc.subcore_barrier()

        base0 = s * per_tile

        def chunk(g, _):
            b = pl.multiple_of(base0 + g * W, W)
            pltpu.sync_copy(srcp.at[pl.ds(b, W)], isrc)
            pltpu.sync_copy(dstp.at[pl.ds(b, W)], idst)
            for q in range(W // 16):
                sl = pl.ds(q * 16, 16)
                i2s[sl] = isrc[sl] * 2 + c
                i2d[sl] = idst[sl] * 2 + c
            g1 = pltpu.async_copy(ex2.at[i2s], rE, sem)
            g2 = pltpu.async_copy(dx2.at[i2d], rD, sem)
            g3 = pltpu.async_copy(bx2.at[i2s], rB, sem)
            g1.wait()
            g2.wait()
            g3.wait()

            def ew(w, _):
                for f in range(HALF // 16):
                    sl = pl.ds(f * 16, 16)
                    t = rD[w, sl] + rE[w, sl]
                    sg = 1.0 / (1.0 + jnp.exp(-t))
                    cd[w, sl] = sg
                    cn[w, sl] = sg * rB[w, sl]
                return 0

            lax.fori_loop(0, W, ew, 0)
            pltpu.sync_copy(cn, accn.at[idst], add=True)
            pltpu.sync_copy(cd, accd.at[idst], add=True)
            return 0

        lax.fori_loop(0, chunks, chunk, 0)
        plsc.subcore_barrier()

        for k in range(rows_pt // 64):
            rr = pl.ds(r0 + k * 64, 64)
            pltpu.sync_copy(accn.at[rr], num_out.at[rr, c])
            pltpu.sync_copy(accd.at[rr], den_out.at[rr, c])

    return edge_kernel


def _matmul4(h, wc, bc):
    NP = h.shape[0]

    def body(h_ref, w_ref, b_ref, o0, o1, o2, o3):
        y = jnp.dot(h_ref[...], w_ref[...],
                    preferred_element_type=jnp.float32) + b_ref[...]
        o0[...] = y[:, 0 * D:1 * D]
        o1[...] = y[:, 1 * D:2 * D]
        o2[...] = y[:, 2 * D:3 * D]
        o3[...] = y[:, 3 * D:4 * D]

    out = [jax.ShapeDtypeStruct((NP, D), jnp.float32)] * 4
    return pl.pallas_call(
        body, grid=(NP // NB,),
        in_specs=[pl.BlockSpec((NB, D), lambda i: (i, 0)),
                  pl.BlockSpec((D, 4 * D), lambda i: (0, 0)),
                  pl.BlockSpec((1, 4 * D), lambda i: (0, 0))],
        out_specs=[pl.BlockSpec((NB, D), lambda i: (i, 0))] * 4,
        out_shape=out)(h, wc, bc)


def _stats(ax, num, den, n_valid):
    NP = ax.shape[0]

    def body(a_ref, n_ref, d_ref, s_ref, q_ref):
        i = pl.program_id(0)
        hn = a_ref[...] + n_ref[...] / (d_ref[...] + 1e-6)
        row = lax.broadcasted_iota(jnp.int32, (NB, 1), 0) + i * NB
        hm = jnp.where(row < n_valid, hn, 0.0)

        @pl.when(i == 0)
        def _():
            s_ref[...] = jnp.zeros_like(s_ref)
            q_ref[...] = jnp.zeros_like(q_ref)

        s_ref[...] += jnp.sum(hm, axis=0, keepdims=True)
        q_ref[...] += jnp.sum(hm * hm, axis=0, keepdims=True)

    out = [jax.ShapeDtypeStruct((1, D), jnp.float32)] * 2
    return pl.pallas_call(
        body, grid=(NP // NB,),
        in_specs=[pl.BlockSpec((NB, D), lambda i: (i, 0))] * 3,
        out_specs=[pl.BlockSpec((1, D), lambda i: (0, 0))] * 2,
        out_shape=out)(ax, num, den)


def _apply(x, ax, num, den, sums, sumsq, bn_g, bn_b, n_valid):
    NP = x.shape[0]
    inv_n = 1.0 / n_valid

    def body(x_ref, a_ref, n_ref, d_ref, s_ref, q_ref, g_ref, b_ref, o_ref):
        i = pl.program_id(0)
        hn = a_ref[...] + n_ref[...] / (d_ref[...] + 1e-6)
        mean = s_ref[...] * inv_n
        var = q_ref[...] * inv_n - mean * mean
        y = (hn - mean) * lax.rsqrt(var + 1e-5) * g_ref[...] + b_ref[...]
        h = x_ref[...] + jnp.maximum(y, 0.0)
        row = lax.broadcasted_iota(jnp.int32, (NB, 1), 0) + i * NB
        o_ref[...] = jnp.where(row < n_valid, h, 0.0)

    return pl.pallas_call(
        body, grid=(NP // NB,),
        in_specs=[pl.BlockSpec((NB, D), lambda i: (i, 0))] * 4 +
                 [pl.BlockSpec((1, D), lambda i: (0, 0))] * 4,
        out_specs=pl.BlockSpec((NB, D), lambda i: (i, 0)),
        out_shape=jax.ShapeDtypeStruct((NP, D), jnp.float32),
    )(x, ax, num, den, sums, sumsq, bn_g, bn_b)


def _pool(h, bids):
    NP = h.shape[0]

    def body(h_ref, b_ref, s_ref, c_ref):
        i = pl.program_id(0)
        oh = (b_ref[...] == lax.broadcasted_iota(jnp.int32, (1, G), 1))
        oh = oh.astype(jnp.float32)

        @pl.when(i == 0)
        def _():
            s_ref[...] = jnp.zeros_like(s_ref)
            c_ref[...] = jnp.zeros_like(c_ref)

        dn = (((0,), (0,)), ((), ()))
        s_ref[...] += lax.dot_general(oh, h_ref[...], dn,
                                      preferred_element_type=jnp.float32)
        c_ref[...] += lax.dot_general(oh, jnp.ones((NB, D), jnp.float32), dn,
                                      preferred_element_type=jnp.float32)

    out = [jax.ShapeDtypeStruct((G, D), jnp.float32)] * 2
    return pl.pallas_call(
        body, grid=(NP // NB,),
        in_specs=[pl.BlockSpec((NB, D), lambda i: (i, 0)),
                  pl.BlockSpec((NB, 1), lambda i: (i, 0))],
        out_specs=[pl.BlockSpec((G, D), lambda i: (0, 0))] * 2,
        out_shape=out)(h, bids)


def _head(sums, counts, w1, b1, w2, b2):
    def body(s_ref, c_ref, w1_ref, b1_ref, w2_ref, b2_ref, o_ref):
        gm = s_ref[...] / jnp.maximum(c_ref[...], 1.0)
        t = jnp.dot(gm, w1_ref[...], preferred_element_type=jnp.float32)
        t = jnp.maximum(t + b1_ref[...], 0.0)
        o_ref[...] = jnp.dot(t, w2_ref[...],
                             preferred_element_type=jnp.float32) + b2_ref[...]

    return pl.pallas_call(
        body, out_shape=jax.ShapeDtypeStruct((G, D), jnp.float32),
    )(sums, counts, w1, b1, w2, b2)


def kernel(x, edge_attr, edge_index, batch_ids, params):
    del edge_attr
    n, d = x.shape
    assert d == D
    e = edge_index.shape[1]
    NP = ((n + 1 + NB - 1) // NB) * NB
    EP = ((e + TILES * W - 1) // (TILES * W)) * (TILES * W)

    src = edge_index[0].astype(jnp.int32)
    dst = edge_index[1].astype(jnp.int32)
    pad_e = jnp.full((EP - e,), n, jnp.int32)
    srcp = jnp.concatenate([src, pad_e])
    dstp = jnp.concatenate([dst, pad_e])

    h = jnp.zeros((NP, D), jnp.float32).at[:n].set(x)
    bids = jnp.concatenate(
        [batch_ids.astype(jnp.int32), jnp.full((NP - n,), G, jnp.int32)]
    ).reshape(NP, 1)

    edge_kernel = _make_edge_kernel(NP, EP)

    for l in range(2):
        wc = jnp.concatenate([params[f"{nm}_w{l}"]
                              for nm in ("A", "B", "Dm", "Em")], axis=1)
        bc = jnp.concatenate([params[f"{nm}_b{l}"]
                              for nm in ("A", "B", "Dm", "Em")]).reshape(1, 4 * D)
        ax, bx, dx, ex = _matmul4(h, wc, bc)
        num, den = edge_kernel(srcp, dstp,
                               ex.reshape(2 * NP, HALF),
                               dx.reshape(2 * NP, HALF),
                               bx.reshape(2 * NP, HALF))
        num = num.reshape(NP, D)
        den = den.reshape(NP, D)
        sums, sumsq = _stats(ax, num, den, n)
        h = _apply(h, ax, num, den, sums, sumsq,
                   params[f"bn_g{l}"].reshape(1, D),
                   params[f"bn_b{l}"].reshape(1, D), n)

    psums, pcounts = _pool(h, bids)
    return _head(psums, pcounts,
                 params["post_w1"], params["post_b1"].reshape(1, D),
                 params["post_w2"], params["post_b2"].reshape(1, D))


# speedup vs baseline: 7.2778x; 1.5607x over previous
import functools

import jax
import jax.numpy as jnp
from jax import lax
from jax.experimental import pallas as pl
from jax.experimental.pallas import tpu as pltpu
from jax.experimental.pallas import tpu_sc as plsc

D = 128
G = 64
HALF = 64
W = 64
TILES = 16
NB = 512


@functools.lru_cache(maxsize=None)
def _make_edge_kernel(NP, EP):
    per_tile = EP // TILES
    chunks = per_tile // W
    assert chunks % 2 == 0 and chunks >= 4
    rows_pt = NP // TILES
    mesh = plsc.VectorSubcoreMesh(core_axis_name="c", subcore_axis_name="s")
    out_t = (jax.ShapeDtypeStruct((NP, 2, HALF), jnp.float32),
             jax.ShapeDtypeStruct((NP, 2, HALF), jnp.float32))
    idx_t = pltpu.VMEM((W,), jnp.int32)
    row_t = pltpu.VMEM((W, HALF), jnp.float32)
    scratch = (
        [idx_t] * 4 +
        [idx_t] * 6 +
        [row_t] * 6 +
        [row_t] * 4 +
        [pltpu.VMEM((64, HALF), jnp.float32)] +
        [pltpu.VMEM_SHARED((NP, HALF), jnp.float32)] * 2 +
        [pltpu.SemaphoreType.DMA] * 4
    )

    @functools.partial(
        pl.kernel, out_type=out_t, mesh=mesh, scratch_types=scratch,
        compiler_params=pltpu.CompilerParams(use_tc_tiling_on_sc=False))
    def edge_kernel(srcp, dstp, ex2, dx2, bx2, num_out, den_out,
                    isr0, isr1, idr0, idr1,
                    i2s0, i2s1, i2d0, i2d1, sd0, sd1,
                    rE0, rE1, rD0, rD1, rB0, rB1,
                    cn0, cn1, cd0, cd1, zb,
                    accn, accd, semG0, semG1, semI0, semI1):
        israw = (isr0, isr1)
        idraw = (idr0, idr1)
        i2s = (i2s0, i2s1)
        i2d = (i2d0, i2d1)
        sd = (sd0, sd1)
        rE = (rE0, rE1)
        rD = (rD0, rD1)
        rB = (rB0, rB1)
        cn = (cn0, cn1)
        cd = (cd0, cd1)
        semG = (semG0, semG1)
        semI = (semI0, semI1)

        c = lax.axis_index("c")
        s = lax.axis_index("s")

        zv = jnp.zeros((16,), jnp.float32)

        def zrow(i, _):
            for f in range(HALF // 16):
                zb[i, pl.ds(f * 16, 16)] = zv
            return 0

        lax.fori_loop(0, 64, zrow, 0)
        r0 = s * rows_pt
        for k in range(rows_pt // 64):
            pltpu.sync_copy(zb, accn.at[pl.ds(r0 + k * 64, 64)])
            pltpu.sync_copy(zb, accd.at[pl.ds(r0 + k * 64, 64)])
        plsc.subcore_barrier()

        base0 = s * per_tile

        def prep(b):
            for q in range(W // 16):
                sl = pl.ds(q * 16, 16)
                dv = idraw[b][sl]
                i2s[b][sl] = israw[b][sl] * 2 + c
                i2d[b][sl] = dv * 2 + c
                sd[b][sl] = dv

        def issue_gathers(b):
            pltpu.async_copy(ex2.at[i2s[b]], rE[b], semG[b])
            pltpu.async_copy(dx2.at[i2d[b]], rD[b], semG[b])
            pltpu.async_copy(bx2.at[i2s[b]], rB[b], semG[b])

        def issue_idx(b, g):
            off = pl.multiple_of(base0 + g * W, W)
            pltpu.async_copy(srcp.at[pl.ds(off, W)], israw[b], semI[b])
            pltpu.async_copy(dstp.at[pl.ds(off, W)], idraw[b], semI[b])

        def drain_gathers(b):
            pltpu.make_async_copy(ex2.at[pl.ds(0, W)], rE[b], semG[b]).wait()
            pltpu.make_async_copy(dx2.at[pl.ds(0, W)], rD[b], semG[b]).wait()
            pltpu.make_async_copy(bx2.at[pl.ds(0, W)], rB[b], semG[b]).wait()

        def drain_idx(b):
            pltpu.make_async_copy(srcp.at[pl.ds(0, W)], israw[b], semI[b]).wait()
            pltpu.make_async_copy(dstp.at[pl.ds(0, W)], idraw[b], semI[b]).wait()

        def compute(b):
            def ew(w, _):
                for f in range(HALF // 16):
                    sl = pl.ds(f * 16, 16)
                    t = rD[b][w, sl] + rE[b][w, sl]
                    sg = 1.0 / (1.0 + jnp.exp(-t))
                    cd[b][w, sl] = sg
                    cn[b][w, sl] = sg * rB[b][w, sl]
                return 0

            lax.fori_loop(0, W, ew, 0)

        for b in (0, 1):
            off = pl.multiple_of(base0 + b * W, W)
            pltpu.sync_copy(srcp.at[pl.ds(off, W)], israw[b])
            pltpu.sync_copy(dstp.at[pl.ds(off, W)], idraw[b])
            prep(b)
            issue_gathers(b)
        issue_idx(0, 2)
        issue_idx(1, 3)

        def pair(p, _):
            for b in (0, 1):
                g = p * 2 + b
                drain_gathers(b)
                compute(b)
                pltpu.sync_copy(cn[b], accn.at[sd[b]], add=True)
                pltpu.sync_copy(cd[b], accd.at[sd[b]], add=True)

                @pl.when(g + 2 < chunks)
                def _():
                    drain_idx(b)
                    prep(b)
                    issue_gathers(b)

                @pl.when(g + 4 < chunks)
                def _():
                    issue_idx(b, g + 4)
            return 0

        lax.fori_loop(0, chunks // 2, pair, 0)
        plsc.subcore_barrier()

        for k in range(rows_pt // 64):
            rr = pl.ds(r0 + k * 64, 64)
            pltpu.sync_copy(accn.at[rr], num_out.at[rr, c])
            pltpu.sync_copy(accd.at[rr], den_out.at[rr, c])

    return edge_kernel


def _matmul4(h, wc, bc):
    NP = h.shape[0]

    def body(h_ref, w_ref, b_ref, o0, o1, o2, o3):
        y = jnp.dot(h_ref[...], w_ref[...],
                    preferred_element_type=jnp.float32) + b_ref[...]
        o0[...] = y[:, 0 * D:1 * D]
        o1[...] = y[:, 1 * D:2 * D]
        o2[...] = y[:, 2 * D:3 * D]
        o3[...] = y[:, 3 * D:4 * D]

    out = [jax.ShapeDtypeStruct((NP, D), jnp.float32)] * 4
    return pl.pallas_call(
        body, grid=(NP // NB,),
        in_specs=[pl.BlockSpec((NB, D), lambda i: (i, 0)),
                  pl.BlockSpec((D, 4 * D), lambda i: (0, 0)),
                  pl.BlockSpec((1, 4 * D), lambda i: (0, 0))],
        out_specs=[pl.BlockSpec((NB, D), lambda i: (i, 0))] * 4,
        out_shape=out)(h, wc, bc)


def _stats(ax, num, den, n_valid):
    NP = ax.shape[0]

    def body(a_ref, n_ref, d_ref, s_ref, q_ref):
        i = pl.program_id(0)
        hn = a_ref[...] + n_ref[...] / (d_ref[...] + 1e-6)
        row = lax.broadcasted_iota(jnp.int32, (NB, 1), 0) + i * NB
        hm = jnp.where(row < n_valid, hn, 0.0)

        @pl.when(i == 0)
        def _():
            s_ref[...] = jnp.zeros_like(s_ref)
            q_ref[...] = jnp.zeros_like(q_ref)

        s_ref[...] += jnp.sum(hm, axis=0, keepdims=True)
        q_ref[...] += jnp.sum(hm * hm, axis=0, keepdims=True)

    out = [jax.ShapeDtypeStruct((1, D), jnp.float32)] * 2
    return pl.pallas_call(
        body, grid=(NP // NB,),
        in_specs=[pl.BlockSpec((NB, D), lambda i: (i, 0))] * 3,
        out_specs=[pl.BlockSpec((1, D), lambda i: (0, 0))] * 2,
        out_shape=out)(ax, num, den)


def _apply(x, ax, num, den, sums, sumsq, bn_g, bn_b, n_valid):
    NP = x.shape[0]
    inv_n = 1.0 / n_valid

    def body(x_ref, a_ref, n_ref, d_ref, s_ref, q_ref, g_ref, b_ref, o_ref):
        i = pl.program_id(0)
        hn = a_ref[...] + n_ref[...] / (d_ref[...] + 1e-6)
        mean = s_ref[...] * inv_n
        var = q_ref[...] * inv_n - mean * mean
        y = (hn - mean) * lax.rsqrt(var + 1e-5) * g_ref[...] + b_ref[...]
        h = x_ref[...] + jnp.maximum(y, 0.0)
        row = lax.broadcasted_iota(jnp.int32, (NB, 1), 0) + i * NB
        o_ref[...] = jnp.where(row < n_valid, h, 0.0)

    return pl.pallas_call(
        body, grid=(NP // NB,),
        in_specs=[pl.BlockSpec((NB, D), lambda i: (i, 0))] * 4 +
                 [pl.BlockSpec((1, D), lambda i: (0, 0))] * 4,
        out_specs=pl.BlockSpec((NB, D), lambda i: (i, 0)),
        out_shape=jax.ShapeDtypeStruct((NP, D), jnp.float32),
    )(x, ax, num, den, sums, sumsq, bn_g, bn_b)


def _pool(h, bids):
    NP = h.shape[0]

    def body(h_ref, b_ref, s_ref, c_ref):
        i = pl.program_id(0)
        oh = (b_ref[...] == lax.broadcasted_iota(jnp.int32, (1, G), 1))
        oh = oh.astype(jnp.float32)

        @pl.when(i == 0)
        def _():
            s_ref[...] = jnp.zeros_like(s_ref)
            c_ref[...] = jnp.zeros_like(c_ref)

        dn = (((0,), (0,)), ((), ()))
        s_ref[...] += lax.dot_general(oh, h_ref[...], dn,
                                      preferred_element_type=jnp.float32)
        c_ref[...] += lax.dot_general(oh, jnp.ones((NB, D), jnp.float32), dn,
                                      preferred_element_type=jnp.float32)

    out = [jax.ShapeDtypeStruct((G, D), jnp.float32)] * 2
    return pl.pallas_call(
        body, grid=(NP // NB,),
        in_specs=[pl.BlockSpec((NB, D), lambda i: (i, 0)),
                  pl.BlockSpec((NB, 1), lambda i: (i, 0))],
        out_specs=[pl.BlockSpec((G, D), lambda i: (0, 0))] * 2,
        out_shape=out)(h, bids)


def _head(sums, counts, w1, b1, w2, b2):
    def body(s_ref, c_ref, w1_ref, b1_ref, w2_ref, b2_ref, o_ref):
        gm = s_ref[...] / jnp.maximum(c_ref[...], 1.0)
        t = jnp.dot(gm, w1_ref[...], preferred_element_type=jnp.float32)
        t = jnp.maximum(t + b1_ref[...], 0.0)
        o_ref[...] = jnp.dot(t, w2_ref[...],
                             preferred_element_type=jnp.float32) + b2_ref[...]

    return pl.pallas_call(
        body, out_shape=jax.ShapeDtypeStruct((G, D), jnp.float32),
    )(sums, counts, w1, b1, w2, b2)


def kernel(x, edge_attr, edge_index, batch_ids, params):
    del edge_attr
    n, d = x.shape
    assert d == D
    e = edge_index.shape[1]
    NP = ((n + 1 + NB - 1) // NB) * NB
    EPQ = 2 * TILES * W
    EP = ((e + EPQ - 1) // EPQ) * EPQ

    src = edge_index[0].astype(jnp.int32)
    dst = edge_index[1].astype(jnp.int32)
    pad_e = jnp.full((EP - e,), n, jnp.int32)
    srcp = jnp.concatenate([src, pad_e])
    dstp = jnp.concatenate([dst, pad_e])

    h = jnp.zeros((NP, D), jnp.float32).at[:n].set(x)
    bids = jnp.concatenate(
        [batch_ids.astype(jnp.int32), jnp.full((NP - n,), G, jnp.int32)]
    ).reshape(NP, 1)

    edge_kernel = _make_edge_kernel(NP, EP)

    for l in range(2):
        wc = jnp.concatenate([params[f"{nm}_w{l}"]
                              for nm in ("A", "B", "Dm", "Em")], axis=1)
        bc = jnp.concatenate([params[f"{nm}_b{l}"]
                              for nm in ("A", "B", "Dm", "Em")]).reshape(1, 4 * D)
        ax, bx, dx, ex = _matmul4(h, wc, bc)
        num, den = edge_kernel(srcp, dstp,
                               ex.reshape(2 * NP, HALF),
                               dx.reshape(2 * NP, HALF),
                               bx.reshape(2 * NP, HALF))
        num = num.reshape(NP, D)
        den = den.reshape(NP, D)
        sums, sumsq = _stats(ax, num, den, n)
        h = _apply(h, ax, num, den, sums, sumsq,
                   params[f"bn_g{l}"].reshape(1, D),
                   params[f"bn_b{l}"].reshape(1, D), n)

    psums, pcounts = _pool(h, bids)
    return _head(psums, pcounts,
                 params["post_w1"], params["post_b1"].reshape(1, D),
                 params["post_w2"], params["post_b2"].reshape(1, D))
